# baseline (speedup 1.0000x reference)
import functools

import jax
import jax.numpy as jnp
from jax import lax
from jax.experimental import pallas as pl
from jax.experimental.pallas import tpu as pltpu
from jax.experimental.pallas import tpu_sc as plsc

_VOCAB = 1000000
_EMB = 64
_B = 4096
_L = 200
_N = _B * _L

_NC = 2
_NS = 16
_NW = _NC * _NS
_PER_W = _N // _NW

_SUB = 128
_NSUB = 4
_C = _SUB * _NSUB
_NCHUNK = _PER_W // _C

assert _PER_W % _C == 0


@functools.partial(
    pl.kernel,
    mesh=plsc.VectorSubcoreMesh(core_axis_name="c", subcore_axis_name="s"),
    out_type=jax.ShapeDtypeStruct((_N, _EMB), jnp.float32),
    scratch_types=[
        pltpu.VMEM((_NSUB, _SUB), jnp.int32),
        pltpu.VMEM((_C, _EMB), jnp.float32),
        pltpu.SemaphoreType.DMA,
    ],
    compiler_params=pltpu.CompilerParams(use_tc_tiling_on_sc=False),
)
def _gather_kernel(table_hbm, idx_hbm, out_hbm, idx_v, rows_v, sem):
    wid = lax.axis_index("s") * _NC + lax.axis_index("c")
    base_row = wid * (_PER_W // _SUB)

    def body(i, _):
        row = base_row + i * _NSUB
        off = (base_row * _SUB) + i * _C
        pltpu.sync_copy(idx_hbm.at[pl.ds(row, _NSUB)], idx_v)
        copies = [
            pltpu.async_copy(
                table_hbm.at[idx_v.at[j]],
                rows_v.at[pl.ds(j * _SUB, _SUB)],
                sem,
            )
            for j in range(_NSUB)
        ]
        for cp in copies:
            cp.wait()
        pltpu.sync_copy(rows_v, out_hbm.at[pl.ds(off, _C)])
        return ()

    lax.fori_loop(0, _NCHUNK, body, ())


def kernel(sequences, token_embedding_matrix):
    idx2d = sequences.reshape(_N // _SUB, _SUB).astype(jnp.int32)
    flat = _gather_kernel(token_embedding_matrix, idx2d)
    return flat.reshape(_B, _L, _EMB), _L


# speedup vs baseline: 1.0443x; 1.0443x over previous
import functools

import jax
import jax.numpy as jnp
from jax import lax
from jax.experimental import pallas as pl
from jax.experimental.pallas import tpu as pltpu
from jax.experimental.pallas import tpu_sc as plsc

_VOCAB = 1000000
_EMB = 64
_B = 4096
_L = 200
_N = _B * _L

_NC = 2
_NS = 16
_NW = _NC * _NS
_PER_W = _N // _NW

_SUB = 128
_NSUB = 4
_C = _SUB * _NSUB
_NCHUNK = _PER_W // _C
_IDX_ROWS = _PER_W // _SUB

assert _PER_W % _C == 0 and _NCHUNK % 2 == 0


@functools.partial(
    pl.kernel,
    mesh=plsc.VectorSubcoreMesh(core_axis_name="c", subcore_axis_name="s"),
    out_type=jax.ShapeDtypeStruct((_N, _EMB), jnp.float32),
    scratch_types=[
        pltpu.VMEM((_IDX_ROWS, _SUB), jnp.int32),
        pltpu.VMEM((2 * _C, _EMB), jnp.float32),
        pltpu.SemaphoreType.DMA,
        pltpu.SemaphoreType.DMA,
        pltpu.SemaphoreType.DMA,
        pltpu.SemaphoreType.DMA,
    ],
    compiler_params=pltpu.CompilerParams(use_tc_tiling_on_sc=False),
)
def _gather_kernel(table_hbm, idx_hbm, out_hbm, idx_v, rows_v, sg0, sg1, sw0, sw1):
    wid = lax.axis_index("s") * _NC + lax.axis_index("c")
    base_row = wid * _IDX_ROWS
    base_off = wid * _PER_W
    sem_g = (sg0, sg1)
    sem_w = (sw0, sw1)

    def _gather_descs(g, b):
        return [
            pltpu.make_async_copy(
                table_hbm.at[idx_v.at[g * _NSUB + j]],
                rows_v.at[pl.ds(b * _C + j * _SUB, _SUB)],
                sem_g[b],
            )
            for j in range(_NSUB)
        ]

    def fire_gather(g, b):
        for cp in _gather_descs(g, b):
            cp.start()

    def drain_gather(g, b):
        for cp in _gather_descs(g, b):
            cp.wait()

    def _write_desc(g, b):
        return pltpu.make_async_copy(
            rows_v.at[pl.ds(b * _C, _C)],
            out_hbm.at[pl.ds(base_off + g * _C, _C)],
            sem_w[b],
        )

    def fire_write(g, b):
        _write_desc(g, b).start()

    def drain_write(g, b):
        _write_desc(g, b).wait()

    pltpu.sync_copy(idx_hbm.at[pl.ds(base_row, _IDX_ROWS)], idx_v)

    fire_gather(0, 0)
    fire_gather(1, 1)

    def body(p, _):
        for b in range(2):
            g = 2 * p + b
            drain_gather(g - 2, b)
            fire_write(g - 2, b)
            drain_write(g - 2, b)
            fire_gather(g, b)
        return ()

    lax.fori_loop(1, _NCHUNK // 2, body, ())

    for b in range(2):
        g = _NCHUNK - 2 + b
        drain_gather(g, b)
        fire_write(g, b)
    for b in range(2):
        g = _NCHUNK - 2 + b
        drain_write(g, b)


def kernel(sequences, token_embedding_matrix):
    idx2d = sequences.reshape(_N // _SUB, _SUB).astype(jnp.int32)
    flat = _gather_kernel(token_embedding_matrix, idx2d)
    return flat.reshape(_B, _L, _EMB), _L


# speedup vs baseline: 1.0455x; 1.0011x over previous
import functools

import jax
import jax.numpy as jnp
from jax import lax
from jax.experimental import pallas as pl
from jax.experimental.pallas import tpu as pltpu
from jax.experimental.pallas import tpu_sc as plsc

_VOCAB = 1000000
_EMB = 64
_B = 4096
_L = 200
_N = _B * _L

_NC = 2
_NS = 16
_NW = _NC * _NS
_PER_W = _N // _NW

_C = 800
_NCHUNK = _PER_W // _C

assert _PER_W % _C == 0 and _NCHUNK % 2 == 0 and _C % 8 == 0


@functools.partial(
    pl.kernel,
    mesh=plsc.VectorSubcoreMesh(core_axis_name="c", subcore_axis_name="s"),
    out_type=jax.ShapeDtypeStruct((_N, _EMB), jnp.float32),
    scratch_types=[
        pltpu.VMEM((_PER_W,), jnp.int32),
        pltpu.VMEM((2 * _C, _EMB), jnp.float32),
        pltpu.SemaphoreType.DMA,
        pltpu.SemaphoreType.DMA,
        pltpu.SemaphoreType.DMA,
        pltpu.SemaphoreType.DMA,
    ],
    compiler_params=pltpu.CompilerParams(use_tc_tiling_on_sc=False),
)
def _gather_kernel(table_hbm, idx_hbm, out_hbm, idx_v, rows_v, sg0, sg1, sw0, sw1):
    wid = lax.axis_index("s") * _NC + lax.axis_index("c")
    base = wid * _PER_W
    sem_g = (sg0, sg1)
    sem_w = (sw0, sw1)

    def _gather_desc(g, b):
        return pltpu.make_async_copy(
            table_hbm.at[idx_v.at[pl.ds(g * _C, _C)]],
            rows_v.at[pl.ds(b * _C, _C)],
            sem_g[b],
        )

    def _write_desc(g, b):
        return pltpu.make_async_copy(
            rows_v.at[pl.ds(b * _C, _C)],
            out_hbm.at[pl.ds(base + g * _C, _C)],
            sem_w[b],
        )

    pltpu.sync_copy(idx_hbm.at[pl.ds(base, _PER_W)], idx_v)

    _gather_desc(0, 0).start()
    _gather_desc(1, 1).start()

    def body(p, _):
        for b in range(2):
            g = 2 * p + b
            _gather_desc(g - 2, b).wait()
            _write_desc(g - 2, b).start()
            _write_desc(g - 2, b).wait()
            _gather_desc(g, b).start()
        return ()

    lax.fori_loop(1, _NCHUNK // 2, body, ())

    for b in range(2):
        g = _NCHUNK - 2 + b
        _gather_desc(g, b).wait()
        _write_desc(g, b).start()
    for b in range(2):
        g = _NCHUNK - 2 + b
        _write_desc(g, b).wait()


def kernel(sequences, token_embedding_matrix):
    idx = sequences.reshape(_N).astype(jnp.int32)
    flat = _gather_kernel(token_embedding_matrix, idx)
    return flat.reshape(_B, _L, _EMB), _L
